# Initial kernel scaffold; baseline (speedup 1.0000x reference)
import jax
import jax.numpy as jnp
from jax.experimental import pallas as pl


def kernel(x_u, x_v, edge_index, edge_attribute, i, dummy, Wu, bu, Wv, bv, Wl1, bl1, root, bias):
    raise NotImplementedError("write your pallas kernel here")


# speedup vs baseline: 5.0572x; 5.0572x over previous
import functools

import jax
import jax.numpy as jnp
from jax import lax
from jax.experimental import pallas as pl
from jax.experimental.pallas import tpu as pltpu
from jax.experimental.pallas import tpu_sc as plsc

D = 32
NC = 2
NS = 16
NW = NC * NS
CHUNK = 128
BN = 1000


def _tca_body(xcat_ref, w3_ref, bcat_ref, w12_ref, xr_ref, y12_ref):
    x0 = jnp.dot(xcat_ref[...], w3_ref[...],
                 preferred_element_type=jnp.float32) + bcat_ref[...]
    xr = jnp.maximum(x0, 0.0)
    xr_ref[...] = xr
    y12_ref[...] = jnp.dot(xr, w12_ref[...], preferred_element_type=jnp.float32)


def _tcb_body(p0_ref, p1_ref, d0_ref, d1_ref, xprev_ref, root_ref, bias_ref,
              w12_ref, xr_ref, y12_ref):
    deg = jnp.maximum(d0_ref[:, 0:1] + d1_ref[:, 0:1], 1.0)
    agg = (p0_ref[...] + p1_ref[...]) / deg
    x1 = agg + jnp.dot(xprev_ref[...], root_ref[...],
                       preferred_element_type=jnp.float32) + bias_ref[...]
    xr = jnp.maximum(x1, 0.0)
    xr_ref[...] = xr
    y12_ref[...] = jnp.dot(xr, w12_ref[...], preferred_element_type=jnp.float32)


def _tcc_body(p0_ref, p1_ref, d0_ref, d1_ref, xprev_ref, root_ref, bias_ref,
              out_ref):
    deg = jnp.maximum(d0_ref[:, 0:1] + d1_ref[:, 0:1], 1.0)
    agg = (p0_ref[...] + p1_ref[...]) / deg
    out_ref[...] = agg + jnp.dot(xprev_ref[...], root_ref[...],
                                 preferred_element_type=jnp.float32) + bias_ref[...]


def _tca(xcat, w3, bcat, w12):
    n = xcat.shape[0]
    grid = (n // BN,)
    return pl.pallas_call(
        _tca_body,
        grid=grid,
        in_specs=[
            pl.BlockSpec((BN, 8), lambda i: (i, 0)),
            pl.BlockSpec((8, D), lambda i: (0, 0)),
            pl.BlockSpec((BN, D), lambda i: (i, 0)),
            pl.BlockSpec((D, 2 * D), lambda i: (0, 0)),
        ],
        out_specs=[pl.BlockSpec((BN, D), lambda i: (i, 0)),
                   pl.BlockSpec((BN, 2 * D), lambda i: (i, 0))],
        out_shape=[jax.ShapeDtypeStruct((n, D), jnp.float32),
                   jax.ShapeDtypeStruct((n, 2 * D), jnp.float32)],
    )(xcat, w3, bcat, w12)


def _tcb(p0, p1, d0, d1, xprev, root, bias2d, w12):
    n = p0.shape[0]
    grid = (n // BN,)
    return pl.pallas_call(
        _tcb_body,
        grid=grid,
        in_specs=[
            pl.BlockSpec((BN, D), lambda i: (i, 0)),
            pl.BlockSpec((BN, D), lambda i: (i, 0)),
            pl.BlockSpec((BN, 16), lambda i: (i, 0)),
            pl.BlockSpec((BN, 16), lambda i: (i, 0)),
            pl.BlockSpec((BN, D), lambda i: (i, 0)),
            pl.BlockSpec((D, D), lambda i: (0, 0)),
            pl.BlockSpec((1, D), lambda i: (0, 0)),
            pl.BlockSpec((D, 2 * D), lambda i: (0, 0)),
        ],
        out_specs=[pl.BlockSpec((BN, D), lambda i: (i, 0)),
                   pl.BlockSpec((BN, 2 * D), lambda i: (i, 0))],
        out_shape=[jax.ShapeDtypeStruct((n, D), jnp.float32),
                   jax.ShapeDtypeStruct((n, 2 * D), jnp.float32)],
    )(p0, p1, d0, d1, xprev, root, bias2d, w12)


def _tcc(p0, p1, d0, d1, xprev, root, bias2d):
    n = p0.shape[0]
    grid = (n // BN,)
    return pl.pallas_call(
        _tcc_body,
        grid=grid,
        in_specs=[
            pl.BlockSpec((BN, D), lambda i: (i, 0)),
            pl.BlockSpec((BN, D), lambda i: (i, 0)),
            pl.BlockSpec((BN, 16), lambda i: (i, 0)),
            pl.BlockSpec((BN, 16), lambda i: (i, 0)),
            pl.BlockSpec((BN, D), lambda i: (i, 0)),
            pl.BlockSpec((D, D), lambda i: (0, 0)),
            pl.BlockSpec((1, D), lambda i: (0, 0)),
        ],
        out_specs=pl.BlockSpec((BN, D), lambda i: (i, 0)),
        out_shape=jax.ShapeDtypeStruct((n, D), jnp.float32),
    )(p0, p1, d0, d1, xprev, root, bias2d)


def _sc_edge_call(y12, srcw, dstw, attr16, z32, z16, o16, nagg, nch, with_deg):
    rps = nagg // NS
    nblk = rps // CHUNK
    mesh = plsc.VectorSubcoreMesh(core_axis_name="c", subcore_axis_name="s")
    out_type = [jax.ShapeDtypeStruct((NC * nagg, D), jnp.float32)]
    scratch = [
        pltpu.VMEM((nch, CHUNK), jnp.int32),
        pltpu.VMEM((nch, CHUNK), jnp.int32),
        pltpu.VMEM((CHUNK, 16), jnp.float32),
        pltpu.VMEM((CHUNK, 2 * D), jnp.float32),
        pltpu.VMEM((CHUNK, D), jnp.float32),
        pltpu.VMEM((CHUNK, 16), jnp.float32),
        pltpu.VMEM_SHARED((nagg, D), jnp.float32),
        pltpu.SemaphoreType.DMA,
    ]
    if with_deg:
        out_type.append(jax.ShapeDtypeStruct((NC * nagg, 16), jnp.float32))
        scratch.append(pltpu.VMEM_SHARED((nagg, 16), jnp.float32))

    def body(y12_hbm, src_hbm, dst_hbm, attr_hbm, z32_hbm, z16_hbm, o16_hbm,
             *refs):
        if with_deg:
            (agg_out, deg_out, srcv, dstv, attrv, rowsv, msgv, onesv, aggS,
             sem, degS) = refs
        else:
            (agg_out, srcv, dstv, attrv, rowsv, msgv, onesv, aggS,
             sem) = refs
        c = lax.axis_index("c")
        s = lax.axis_index("s")
        w = c * NS + s

        pltpu.sync_copy(src_hbm.at[w], srcv)
        pltpu.sync_copy(dst_hbm.at[w], dstv)
        if with_deg:
            pltpu.sync_copy(o16_hbm, onesv)

        @pl.loop(0, nblk)
        def _(t):
            base = s * rps + t * CHUNK
            pltpu.sync_copy(z32_hbm, aggS.at[pl.ds(base, CHUNK)])
            if with_deg:
                pltpu.sync_copy(z16_hbm, degS.at[pl.ds(base, CHUNK)])

        plsc.subcore_barrier()

        @pl.loop(0, nch)
        def _(j):
            pltpu.sync_copy(attr_hbm.at[w * nch + j], attrv)
            pltpu.async_copy(y12_hbm.at[srcv.at[j]], rowsv, sem).wait()

            @pl.loop(0, CHUNK)
            def _(k):
                av = attrv[k, 0:16]
                msgv[k, 0:16] = av * rowsv[k, 0:16] + rowsv[k, 32:48]
                msgv[k, 16:32] = av * rowsv[k, 16:32] + rowsv[k, 48:64]

            pltpu.sync_copy(msgv, aggS.at[dstv.at[j]], add=True)
            if with_deg:
                pltpu.sync_copy(onesv, degS.at[dstv.at[j]], add=True)

        plsc.subcore_barrier()

        @pl.loop(0, nblk)
        def _(t):
            base = s * rps + t * CHUNK
            pltpu.sync_copy(aggS.at[pl.ds(base, CHUNK)],
                            agg_out.at[pl.ds(c * nagg + base, CHUNK)])
            if with_deg:
                pltpu.sync_copy(degS.at[pl.ds(base, CHUNK)],
                                deg_out.at[pl.ds(c * nagg + base, CHUNK)])

    fn = pl.kernel(
        body, mesh=mesh, out_type=out_type, scratch_types=scratch,
        compiler_params=pltpu.CompilerParams(use_tc_tiling_on_sc=False))
    return fn(y12, srcw, dstw, attr16, z32, z16, o16)


def kernel(x_u, x_v, edge_index, edge_attribute, i, dummy,
           Wu, bu, Wv, bv, Wl1, bl1, root, bias):
    n_u = x_u.shape[0]
    n_v = x_v.shape[0]
    n = n_u + n_v
    e = edge_index.shape[1]

    w1 = Wl1.reshape(D, D)
    b1 = bl1.reshape(D, D)
    w12 = jnp.concatenate([w1, b1], axis=1)
    w3 = jnp.concatenate([Wu, Wv, jnp.zeros((5, D), jnp.float32)], axis=0)
    bcat = jnp.concatenate([jnp.broadcast_to(bu, (n_u, D)),
                            jnp.broadcast_to(bv, (n_v, D))], axis=0)
    xcat = jnp.concatenate([
        jnp.pad(x_u, ((0, 0), (0, 7))),
        jnp.pad(x_v, ((0, 0), (1, 5))),
    ], axis=0)
    bias2d = bias.reshape(1, D)

    epad = -(-e // (NW * CHUNK)) * (NW * CHUNK)
    nch = epad // (NW * CHUNK)
    pad = epad - e
    nagg = -(-(n + 1) // (NS * CHUNK)) * (NS * CHUNK)
    src = jnp.concatenate([edge_index[0], jnp.zeros((pad,), jnp.int32)])
    dst = jnp.concatenate([edge_index[1], jnp.full((pad,), n, jnp.int32)])
    attr = jnp.concatenate([edge_attribute[:, 0],
                            jnp.zeros((pad,), jnp.float32)])
    srcw = src.reshape(NW, nch, CHUNK)
    dstw = dst.reshape(NW, nch, CHUNK)
    attr16 = jnp.broadcast_to(attr.reshape(NW * nch, CHUNK, 1),
                              (NW * nch, CHUNK, 16))
    z32 = jnp.zeros((CHUNK, D), jnp.float32)
    z16 = jnp.zeros((CHUNK, 16), jnp.float32)
    o16 = jnp.ones((CHUNK, 16), jnp.float32)

    xr0, y12_0 = _tca(xcat, w3, bcat, w12)
    aggf0, degf = _sc_edge_call(y12_0, srcw, dstw, attr16, z32, z16, o16,
                                nagg, nch, True)
    p0, p1 = aggf0[:n], aggf0[nagg:nagg + n]
    d0, d1 = degf[:n], degf[nagg:nagg + n]
    xr1, y12_1 = _tcb(p0, p1, d0, d1, xr0, root, bias2d, w12)
    aggf1 = _sc_edge_call(y12_1, srcw, dstw, attr16, z32, z16, o16,
                          nagg, nch, False)[0]
    q0, q1 = aggf1[:n], aggf1[nagg:nagg + n]
    return _tcc(q0, q1, d0, d1, xr1, root, bias2d)


# speedup vs baseline: 7.8771x; 1.5576x over previous
import functools

import jax
import jax.numpy as jnp
from jax import lax
from jax.experimental import pallas as pl
from jax.experimental.pallas import tpu as pltpu
from jax.experimental.pallas import tpu_sc as plsc

D = 32
NC = 2
NS = 16
NW = NC * NS
CHUNK = 128
DEPTH = 4
BN = 1000


def _tca_body(xcat_ref, w3_ref, bcat_ref, w12_ref, xr_ref, y12_ref):
    x0 = jnp.dot(xcat_ref[...], w3_ref[...],
                 preferred_element_type=jnp.float32) + bcat_ref[0]
    xr = jnp.maximum(x0, 0.0)
    xr_ref[...] = xr
    y12_ref[...] = jnp.dot(xr, w12_ref[...], preferred_element_type=jnp.float32)


def _tcb_body(p0_ref, p1_ref, d0_ref, d1_ref, xprev_ref, root_ref, bias_ref,
              w12_ref, xr_ref, y12_ref):
    deg = jnp.maximum(d0_ref[0, :, 0:1] + d1_ref[0, :, 0:1], 1.0)
    agg = (p0_ref[0] + p1_ref[0]) / deg
    x1 = agg + jnp.dot(xprev_ref[...], root_ref[...],
                       preferred_element_type=jnp.float32) + bias_ref[...]
    xr = jnp.maximum(x1, 0.0)
    xr_ref[...] = xr
    y12_ref[...] = jnp.dot(xr, w12_ref[...], preferred_element_type=jnp.float32)


def _tcc_body(p0_ref, p1_ref, d0_ref, d1_ref, xprev_ref, root_ref, bias_ref,
              out_ref):
    deg = jnp.maximum(d0_ref[0, :, 0:1] + d1_ref[0, :, 0:1], 1.0)
    agg = (p0_ref[0] + p1_ref[0]) / deg
    out_ref[...] = agg + jnp.dot(xprev_ref[...], root_ref[...],
                                 preferred_element_type=jnp.float32) + bias_ref[...]


def _tca(xcat, w3, bcat2, w12):
    n = xcat.shape[0]
    grid = (n // BN,)
    return pl.pallas_call(
        _tca_body,
        grid=grid,
        in_specs=[
            pl.BlockSpec((BN, 8), lambda i: (i, 0)),
            pl.BlockSpec((8, D), lambda i: (0, 0)),
            pl.BlockSpec((1, 1, D), lambda i: (i * BN * 2 // n, 0, 0)),
            pl.BlockSpec((D, 2 * D), lambda i: (0, 0)),
        ],
        out_specs=[pl.BlockSpec((BN, D), lambda i: (i, 0)),
                   pl.BlockSpec((BN, 2 * D), lambda i: (i, 0))],
        out_shape=[jax.ShapeDtypeStruct((n, D), jnp.float32),
                   jax.ShapeDtypeStruct((n, 2 * D), jnp.float32)],
    )(xcat, w3, bcat2, w12)


def _agg_specs():
    return [
        pl.BlockSpec((1, BN, D), lambda i: (0, i, 0)),
        pl.BlockSpec((1, BN, D), lambda i: (1, i, 0)),
        pl.BlockSpec((1, BN, 16), lambda i: (0, i, 0)),
        pl.BlockSpec((1, BN, 16), lambda i: (1, i, 0)),
    ]


def _tcb(aggf, degf, xprev, root, bias2d, w12):
    n = xprev.shape[0]
    grid = (n // BN,)
    return pl.pallas_call(
        _tcb_body,
        grid=grid,
        in_specs=_agg_specs() + [
            pl.BlockSpec((BN, D), lambda i: (i, 0)),
            pl.BlockSpec((D, D), lambda i: (0, 0)),
            pl.BlockSpec((1, D), lambda i: (0, 0)),
            pl.BlockSpec((D, 2 * D), lambda i: (0, 0)),
        ],
        out_specs=[pl.BlockSpec((BN, D), lambda i: (i, 0)),
                   pl.BlockSpec((BN, 2 * D), lambda i: (i, 0))],
        out_shape=[jax.ShapeDtypeStruct((n, D), jnp.float32),
                   jax.ShapeDtypeStruct((n, 2 * D), jnp.float32)],
    )(aggf, aggf, degf, degf, xprev, root, bias2d, w12)


def _tcc(aggf, degf, xprev, root, bias2d):
    n = xprev.shape[0]
    grid = (n // BN,)
    return pl.pallas_call(
        _tcc_body,
        grid=grid,
        in_specs=_agg_specs() + [
            pl.BlockSpec((BN, D), lambda i: (i, 0)),
            pl.BlockSpec((D, D), lambda i: (0, 0)),
            pl.BlockSpec((1, D), lambda i: (0, 0)),
        ],
        out_specs=pl.BlockSpec((BN, D), lambda i: (i, 0)),
        out_shape=jax.ShapeDtypeStruct((n, D), jnp.float32),
    )(aggf, aggf, degf, degf, xprev, root, bias2d)


def _sc_edge_call(y12, srcw, dstw, attr16, z32, z16, o16, nagg, nch, with_deg):
    rps = nagg // NS
    nblk = rps // CHUNK
    mesh = plsc.VectorSubcoreMesh(core_axis_name="c", subcore_axis_name="s")
    out_type = [jax.ShapeDtypeStruct((NC, nagg, D), jnp.float32)]
    scratch = [
        pltpu.VMEM((nch, CHUNK), jnp.int32),
        pltpu.VMEM((nch, CHUNK), jnp.int32),
        pltpu.VMEM((DEPTH, CHUNK, 16), jnp.float32),
        pltpu.VMEM((DEPTH, CHUNK, 2 * D), jnp.float32),
        pltpu.VMEM((CHUNK, D), jnp.float32),
        pltpu.VMEM((CHUNK, 16), jnp.float32),
        pltpu.VMEM_SHARED((nagg, D), jnp.float32),
        pltpu.SemaphoreType.DMA((DEPTH,)),
        pltpu.SemaphoreType.DMA((DEPTH,)),
    ]
    if with_deg:
        out_type.append(jax.ShapeDtypeStruct((NC, nagg, 16), jnp.float32))
        scratch.append(pltpu.VMEM_SHARED((nagg, 16), jnp.float32))

    def body(y12_hbm, src_hbm, dst_hbm, attr_hbm, z32_hbm, z16_hbm, o16_hbm,
             *refs):
        if with_deg:
            (agg_out, deg_out, srcv, dstv, attrv, rowsv, msgv, onesv, aggS,
             sem_a, sem_g, degS) = refs
        else:
            (agg_out, srcv, dstv, attrv, rowsv, msgv, onesv, aggS,
             sem_a, sem_g) = refs
        c = lax.axis_index("c")
        s = lax.axis_index("s")
        w = c * NS + s

        pltpu.sync_copy(src_hbm.at[w], srcv)
        pltpu.sync_copy(dst_hbm.at[w], dstv)
        if with_deg:
            pltpu.sync_copy(o16_hbm, onesv)

        def start(j, b):
            pltpu.async_copy(attr_hbm.at[w * nch + j], attrv.at[b],
                             sem_a.at[b])
            pltpu.async_copy(y12_hbm.at[srcv.at[j]], rowsv.at[b],
                             sem_g.at[b])

        @pl.loop(0, nblk)
        def _(t):
            base = s * rps + t * CHUNK
            pltpu.sync_copy(z32_hbm, aggS.at[pl.ds(base, CHUNK)])
            if with_deg:
                pltpu.sync_copy(z16_hbm, degS.at[pl.ds(base, CHUNK)])

        plsc.subcore_barrier()

        for b in range(DEPTH):
            start(b, b)

        @pl.loop(0, nch)
        def _(j):
            b = lax.rem(j, DEPTH)
            pltpu.make_async_copy(attr_hbm.at[0], attrv.at[b],
                                  sem_a.at[b]).wait()
            pltpu.make_async_copy(y12_hbm.at[srcv.at[0]], rowsv.at[b],
                                  sem_g.at[b]).wait()
            rv = rowsv.at[b]
            av_ = attrv.at[b]

            @pl.loop(0, CHUNK)
            def _(k):
                av = av_[k, 0:16]
                msgv[k, 0:16] = av * rv[k, 0:16] + rv[k, 32:48]
                msgv[k, 16:32] = av * rv[k, 16:32] + rv[k, 48:64]

            @pl.when(j + DEPTH < nch)
            def _():
                start(j + DEPTH, b)

            pltpu.sync_copy(msgv, aggS.at[dstv.at[j]], add=True)
            if with_deg:
                pltpu.sync_copy(onesv, degS.at[dstv.at[j]], add=True)

        plsc.subcore_barrier()

        @pl.loop(0, nblk)
        def _(t):
            base = s * rps + t * CHUNK
            pltpu.sync_copy(aggS.at[pl.ds(base, CHUNK)],
                            agg_out.at[c].at[pl.ds(base, CHUNK)])
            if with_deg:
                pltpu.sync_copy(degS.at[pl.ds(base, CHUNK)],
                                deg_out.at[c].at[pl.ds(base, CHUNK)])

    fn = pl.kernel(
        body, mesh=mesh, out_type=out_type, scratch_types=scratch,
        compiler_params=pltpu.CompilerParams(use_tc_tiling_on_sc=False))
    return fn(y12, srcw, dstw, attr16, z32, z16, o16)


def kernel(x_u, x_v, edge_index, edge_attribute, i, dummy,
           Wu, bu, Wv, bv, Wl1, bl1, root, bias):
    n_u = x_u.shape[0]
    n_v = x_v.shape[0]
    n = n_u + n_v
    e = edge_index.shape[1]

    w1 = Wl1.reshape(D, D)
    b1 = bl1.reshape(D, D)
    w12 = jnp.concatenate([w1, b1], axis=1)
    w3 = jnp.concatenate([Wu, Wv, jnp.zeros((5, D), jnp.float32)], axis=0)
    bcat2 = jnp.stack([bu, bv], axis=0).reshape(2, 1, D)
    xcat = jnp.concatenate([
        jnp.pad(x_u, ((0, 0), (0, 7))),
        jnp.pad(x_v, ((0, 0), (1, 5))),
    ], axis=0)
    bias2d = bias.reshape(1, D)

    epad = -(-e // (NW * CHUNK)) * (NW * CHUNK)
    nch = epad // (NW * CHUNK)
    pad = epad - e
    nagg = -(-(n + 1) // (NS * CHUNK)) * (NS * CHUNK)
    src = jnp.concatenate([edge_index[0], jnp.zeros((pad,), jnp.int32)])
    dst = jnp.concatenate([edge_index[1], jnp.full((pad,), n, jnp.int32)])
    attr = jnp.concatenate([edge_attribute[:, 0],
                            jnp.zeros((pad,), jnp.float32)])
    srcw = src.reshape(NW, nch, CHUNK)
    dstw = dst.reshape(NW, nch, CHUNK)
    attr16 = jnp.broadcast_to(attr.reshape(NW * nch, CHUNK, 1),
                              (NW * nch, CHUNK, 16))
    z32 = jnp.zeros((CHUNK, D), jnp.float32)
    z16 = jnp.zeros((CHUNK, 16), jnp.float32)
    o16 = jnp.ones((CHUNK, 16), jnp.float32)

    xr0, y12_0 = _tca(xcat, w3, bcat2, w12)
    aggf0, degf = _sc_edge_call(y12_0, srcw, dstw, attr16, z32, z16, o16,
                                nagg, nch, True)
    xr1, y12_1 = _tcb(aggf0, degf, xr0, root, bias2d, w12)
    aggf1 = _sc_edge_call(y12_1, srcw, dstw, attr16, z32, z16, o16,
                          nagg, nch, False)[0]
    return _tcc(aggf1, degf, xr1, root, bias2d)


# speedup vs baseline: 7.9067x; 1.0038x over previous
import functools

import jax
import jax.numpy as jnp
from jax import lax
from jax.experimental import pallas as pl
from jax.experimental.pallas import tpu as pltpu
from jax.experimental.pallas import tpu_sc as plsc

D = 32
NC = 2
NS = 16
NW = NC * NS
CHUNK = 128
DEPTH = 4
BN = 1000


def _tca_body(xcat_ref, w3_ref, bcat_ref, w12_ref, xr_ref, y12_ref):
    x0 = jnp.dot(xcat_ref[...], w3_ref[...],
                 preferred_element_type=jnp.float32) + bcat_ref[0]
    xr = jnp.maximum(x0, 0.0)
    xr_ref[...] = xr
    y12_ref[...] = jnp.dot(xr, w12_ref[...], preferred_element_type=jnp.float32)


def _tcb_body(p0_ref, p1_ref, d0_ref, d1_ref, xprev_ref, root_ref, bias_ref,
              w12_ref, xr_ref, y12_ref):
    deg = jnp.maximum(d0_ref[0, :, 0:1] + d1_ref[0, :, 0:1], 1.0)
    agg = (p0_ref[0] + p1_ref[0]) / deg
    x1 = agg + jnp.dot(xprev_ref[...], root_ref[...],
                       preferred_element_type=jnp.float32) + bias_ref[...]
    xr = jnp.maximum(x1, 0.0)
    xr_ref[...] = xr
    y12_ref[...] = jnp.dot(xr, w12_ref[...], preferred_element_type=jnp.float32)


def _tcc_body(p0_ref, p1_ref, d0_ref, d1_ref, xprev_ref, root_ref, bias_ref,
              out_ref):
    deg = jnp.maximum(d0_ref[0, :, 0:1] + d1_ref[0, :, 0:1], 1.0)
    agg = (p0_ref[0] + p1_ref[0]) / deg
    out_ref[...] = agg + jnp.dot(xprev_ref[...], root_ref[...],
                                 preferred_element_type=jnp.float32) + bias_ref[...]


def _tca(xcat, w3, bcat2, w12):
    n = xcat.shape[0]
    grid = (n // BN,)
    return pl.pallas_call(
        _tca_body,
        grid=grid,
        in_specs=[
            pl.BlockSpec((BN, 8), lambda i: (i, 0)),
            pl.BlockSpec((8, D), lambda i: (0, 0)),
            pl.BlockSpec((1, 1, D), lambda i: (i * BN * 2 // n, 0, 0)),
            pl.BlockSpec((D, 2 * D), lambda i: (0, 0)),
        ],
        out_specs=[pl.BlockSpec((BN, D), lambda i: (i, 0)),
                   pl.BlockSpec((BN, 2 * D), lambda i: (i, 0))],
        out_shape=[jax.ShapeDtypeStruct((n, D), jnp.float32),
                   jax.ShapeDtypeStruct((n, 2 * D), jnp.float32)],
    )(xcat, w3, bcat2, w12)


def _agg_specs():
    return [
        pl.BlockSpec((1, BN, D), lambda i: (0, i, 0)),
        pl.BlockSpec((1, BN, D), lambda i: (1, i, 0)),
        pl.BlockSpec((1, BN, 16), lambda i: (0, i, 0)),
        pl.BlockSpec((1, BN, 16), lambda i: (1, i, 0)),
    ]


def _tcb(aggf, degf, xprev, root, bias2d, w12):
    n = xprev.shape[0]
    grid = (n // BN,)
    return pl.pallas_call(
        _tcb_body,
        grid=grid,
        in_specs=_agg_specs() + [
            pl.BlockSpec((BN, D), lambda i: (i, 0)),
            pl.BlockSpec((D, D), lambda i: (0, 0)),
            pl.BlockSpec((1, D), lambda i: (0, 0)),
            pl.BlockSpec((D, 2 * D), lambda i: (0, 0)),
        ],
        out_specs=[pl.BlockSpec((BN, D), lambda i: (i, 0)),
                   pl.BlockSpec((BN, 2 * D), lambda i: (i, 0))],
        out_shape=[jax.ShapeDtypeStruct((n, D), jnp.float32),
                   jax.ShapeDtypeStruct((n, 2 * D), jnp.float32)],
    )(aggf, aggf, degf, degf, xprev, root, bias2d, w12)


def _tcc(aggf, degf, xprev, root, bias2d):
    n = xprev.shape[0]
    grid = (n // BN,)
    return pl.pallas_call(
        _tcc_body,
        grid=grid,
        in_specs=_agg_specs() + [
            pl.BlockSpec((BN, D), lambda i: (i, 0)),
            pl.BlockSpec((D, D), lambda i: (0, 0)),
            pl.BlockSpec((1, D), lambda i: (0, 0)),
        ],
        out_specs=pl.BlockSpec((BN, D), lambda i: (i, 0)),
        out_shape=jax.ShapeDtypeStruct((n, D), jnp.float32),
    )(aggf, aggf, degf, degf, xprev, root, bias2d)


def _sc_edge_call(y12, srcw, dstw, attr16, z32, z16, o16, nagg, nch, with_deg):
    rps = nagg // NS
    nblk = rps // CHUNK
    mesh = plsc.VectorSubcoreMesh(core_axis_name="c", subcore_axis_name="s")
    out_type = [jax.ShapeDtypeStruct((NC, nagg, D), jnp.float32)]
    scratch = [
        pltpu.VMEM((nch, CHUNK), jnp.int32),
        pltpu.VMEM((nch, CHUNK), jnp.int32),
        pltpu.VMEM((DEPTH, CHUNK, 16), jnp.float32),
        pltpu.VMEM((DEPTH, CHUNK, 2 * D), jnp.float32),
        pltpu.VMEM((2, CHUNK, D), jnp.float32),
        pltpu.VMEM((CHUNK, 16), jnp.float32),
        pltpu.VMEM_SHARED((nagg, D), jnp.float32),
        pltpu.SemaphoreType.DMA((DEPTH,)),
        pltpu.SemaphoreType.DMA((DEPTH,)),
        pltpu.SemaphoreType.DMA((2,)),
        pltpu.SemaphoreType.DMA((2,)),
    ]
    if with_deg:
        out_type.append(jax.ShapeDtypeStruct((NC, nagg, 16), jnp.float32))
        scratch.append(pltpu.VMEM_SHARED((nagg, 16), jnp.float32))

    def body(y12_hbm, src_hbm, dst_hbm, attr_hbm, z32_hbm, z16_hbm, o16_hbm,
             *refs):
        if with_deg:
            (agg_out, deg_out, srcv, dstv, attrv, rowsv, msgv, onesv, aggS,
             sem_a, sem_g, sem_s, sem_d, degS) = refs
        else:
            (agg_out, srcv, dstv, attrv, rowsv, msgv, onesv, aggS,
             sem_a, sem_g, sem_s, sem_d) = refs
        c = lax.axis_index("c")
        s = lax.axis_index("s")
        w = c * NS + s

        pltpu.sync_copy(src_hbm.at[w], srcv)
        pltpu.sync_copy(dst_hbm.at[w], dstv)
        if with_deg:
            pltpu.sync_copy(o16_hbm, onesv)

        def start(j, b):
            pltpu.async_copy(attr_hbm.at[w * nch + j], attrv.at[b],
                             sem_a.at[b])
            pltpu.async_copy(y12_hbm.at[srcv.at[j]], rowsv.at[b],
                             sem_g.at[b])

        @pl.loop(0, nblk)
        def _(t):
            base = s * rps + t * CHUNK
            pltpu.sync_copy(z32_hbm, aggS.at[pl.ds(base, CHUNK)])
            if with_deg:
                pltpu.sync_copy(z16_hbm, degS.at[pl.ds(base, CHUNK)])

        plsc.subcore_barrier()

        for b in range(DEPTH):
            start(b, b)

        @pl.loop(0, nch)
        def _(j):
            b = lax.rem(j, DEPTH)
            mb = lax.rem(j, 2)
            pltpu.make_async_copy(attr_hbm.at[0], attrv.at[b],
                                  sem_a.at[b]).wait()
            pltpu.make_async_copy(y12_hbm.at[srcv.at[0]], rowsv.at[b],
                                  sem_g.at[b]).wait()
            rv = rowsv.at[b]
            av_ = attrv.at[b]
            mv = msgv.at[mb]

            @pl.when(j >= 2)
            def _():
                pltpu.make_async_copy(msgv.at[mb], aggS.at[dstv.at[0]],
                                      sem_s.at[mb]).wait()
                if with_deg:
                    pltpu.make_async_copy(onesv, degS.at[dstv.at[0]],
                                          sem_d.at[mb]).wait()

            @pl.loop(0, CHUNK)
            def _(k):
                av = av_[k, 0:16]
                mv[k, 0:16] = av * rv[k, 0:16] + rv[k, 32:48]
                mv[k, 16:32] = av * rv[k, 16:32] + rv[k, 48:64]

            @pl.when(j + DEPTH < nch)
            def _():
                start(j + DEPTH, b)

            pltpu.async_copy(msgv.at[mb], aggS.at[dstv.at[j]], sem_s.at[mb],
                             add=True)
            if with_deg:
                pltpu.async_copy(onesv, degS.at[dstv.at[j]], sem_d.at[mb],
                                 add=True)

        for mb in range(2):
            pltpu.make_async_copy(msgv.at[mb], aggS.at[dstv.at[0]],
                                  sem_s.at[mb]).wait()
            if with_deg:
                pltpu.make_async_copy(onesv, degS.at[dstv.at[0]],
                                      sem_d.at[mb]).wait()

        plsc.subcore_barrier()

        @pl.loop(0, nblk)
        def _(t):
            base = s * rps + t * CHUNK
            pltpu.sync_copy(aggS.at[pl.ds(base, CHUNK)],
                            agg_out.at[c].at[pl.ds(base, CHUNK)])
            if with_deg:
                pltpu.sync_copy(degS.at[pl.ds(base, CHUNK)],
                                deg_out.at[c].at[pl.ds(base, CHUNK)])

    fn = pl.kernel(
        body, mesh=mesh, out_type=out_type, scratch_types=scratch,
        compiler_params=pltpu.CompilerParams(use_tc_tiling_on_sc=False))
    return fn(y12, srcw, dstw, attr16, z32, z16, o16)


def kernel(x_u, x_v, edge_index, edge_attribute, i, dummy,
           Wu, bu, Wv, bv, Wl1, bl1, root, bias):
    n_u = x_u.shape[0]
    n_v = x_v.shape[0]
    n = n_u + n_v
    e = edge_index.shape[1]

    w1 = Wl1.reshape(D, D)
    b1 = bl1.reshape(D, D)
    w12 = jnp.concatenate([w1, b1], axis=1)
    w3 = jnp.concatenate([Wu, Wv, jnp.zeros((5, D), jnp.float32)], axis=0)
    bcat2 = jnp.stack([bu, bv], axis=0).reshape(2, 1, D)
    xcat = jnp.concatenate([
        jnp.pad(x_u, ((0, 0), (0, 7))),
        jnp.pad(x_v, ((0, 0), (1, 5))),
    ], axis=0)
    bias2d = bias.reshape(1, D)

    epad = -(-e // (NW * CHUNK)) * (NW * CHUNK)
    nch = epad // (NW * CHUNK)
    pad = epad - e
    nagg = -(-(n + 1) // (NS * CHUNK)) * (NS * CHUNK)
    src = jnp.concatenate([edge_index[0], jnp.zeros((pad,), jnp.int32)])
    dst = jnp.concatenate([edge_index[1], jnp.full((pad,), n, jnp.int32)])
    attr = jnp.concatenate([edge_attribute[:, 0],
                            jnp.zeros((pad,), jnp.float32)])
    srcw = src.reshape(NW, nch, CHUNK)
    dstw = dst.reshape(NW, nch, CHUNK)
    attr16 = jnp.broadcast_to(attr.reshape(NW * nch, CHUNK, 1),
                              (NW * nch, CHUNK, 16))
    z32 = jnp.zeros((CHUNK, D), jnp.float32)
    z16 = jnp.zeros((CHUNK, 16), jnp.float32)
    o16 = jnp.ones((CHUNK, 16), jnp.float32)

    xr0, y12_0 = _tca(xcat, w3, bcat2, w12)
    aggf0, degf = _sc_edge_call(y12_0, srcw, dstw, attr16, z32, z16, o16,
                                nagg, nch, True)
    xr1, y12_1 = _tcb(aggf0, degf, xr0, root, bias2d, w12)
    aggf1 = _sc_edge_call(y12_1, srcw, dstw, attr16, z32, z16, o16,
                          nagg, nch, False)[0]
    return _tcc(aggf1, degf, xr1, root, bias2d)


# speedup vs baseline: 9.9225x; 1.2549x over previous
import functools

import jax
import jax.numpy as jnp
from jax import lax
from jax.experimental import pallas as pl
from jax.experimental.pallas import tpu as pltpu
from jax.experimental.pallas import tpu_sc as plsc

D = 32
NC = 2
NS = 16
NW = NC * NS
CHUNK = 128
DEPTH = 4
BN = 1000


def _make_tca_body(n_u):
    def _tca_body(xcat_ref, w3_ref, bcat_ref, w12_ref, xr_ref, y12_ref):
        n = xcat_ref.shape[0]
        row = jax.lax.broadcasted_iota(jnp.int32, (n, 1), 0)
        b = jnp.where(row < n_u, bcat_ref[0], bcat_ref[1])
        x0 = jnp.dot(xcat_ref[...], w3_ref[...],
                     preferred_element_type=jnp.float32) + b
        xr = jnp.maximum(x0, 0.0)
        xr_ref[...] = xr
        y12_ref[...] = jnp.dot(xr, w12_ref[...],
                               preferred_element_type=jnp.float32)
    return _tca_body


def _tcb_body(p0_ref, p1_ref, d0_ref, d1_ref, xprev_ref, root_ref, bias_ref,
              w12_ref, xr_ref, y12_ref):
    deg = jnp.maximum(d0_ref[0, :, 0:1] + d1_ref[0, :, 0:1], 1.0)
    agg = (p0_ref[0] + p1_ref[0]) / deg
    x1 = agg + jnp.dot(xprev_ref[...], root_ref[...],
                       preferred_element_type=jnp.float32) + bias_ref[...]
    xr = jnp.maximum(x1, 0.0)
    xr_ref[...] = xr
    y12_ref[...] = jnp.dot(xr, w12_ref[...], preferred_element_type=jnp.float32)


def _tcc_body(p0_ref, p1_ref, d0_ref, d1_ref, xprev_ref, root_ref, bias_ref,
              out_ref):
    deg = jnp.maximum(d0_ref[0, :, 0:1] + d1_ref[0, :, 0:1], 1.0)
    agg = (p0_ref[0] + p1_ref[0]) / deg
    out_ref[...] = agg + jnp.dot(xprev_ref[...], root_ref[...],
                                 preferred_element_type=jnp.float32) + bias_ref[...]


def _tca(xcat, w3, bcat2, w12, n_u):
    n = xcat.shape[0]
    return pl.pallas_call(
        _make_tca_body(n_u),
        out_shape=[jax.ShapeDtypeStruct((n, D), jnp.float32),
                   jax.ShapeDtypeStruct((n, 2 * D), jnp.float32)],
    )(xcat, w3, bcat2, w12)


def _agg_specs(n, nagg):
    return [
        pl.BlockSpec((1, n, D), lambda i: (0, 0, 0)),
        pl.BlockSpec((1, n, D), lambda i: (1, 0, 0)),
        pl.BlockSpec((1, n, 16), lambda i: (0, 0, 0)),
        pl.BlockSpec((1, n, 16), lambda i: (1, 0, 0)),
    ]


def _tcb(aggf, degf, xprev, root, bias2d, w12):
    n = xprev.shape[0]
    nagg = aggf.shape[1]
    return pl.pallas_call(
        _tcb_body,
        grid=(1,),
        in_specs=_agg_specs(n, nagg) + [
            pl.BlockSpec((n, D), lambda i: (0, 0)),
            pl.BlockSpec((D, D), lambda i: (0, 0)),
            pl.BlockSpec((1, D), lambda i: (0, 0)),
            pl.BlockSpec((D, 2 * D), lambda i: (0, 0)),
        ],
        out_specs=[pl.BlockSpec((n, D), lambda i: (0, 0)),
                   pl.BlockSpec((n, 2 * D), lambda i: (0, 0))],
        out_shape=[jax.ShapeDtypeStruct((n, D), jnp.float32),
                   jax.ShapeDtypeStruct((n, 2 * D), jnp.float32)],
    )(aggf, aggf, degf, degf, xprev, root, bias2d, w12)


def _tcc(aggf, degf, xprev, root, bias2d):
    n = xprev.shape[0]
    nagg = aggf.shape[1]
    return pl.pallas_call(
        _tcc_body,
        grid=(1,),
        in_specs=_agg_specs(n, nagg) + [
            pl.BlockSpec((n, D), lambda i: (0, 0)),
            pl.BlockSpec((D, D), lambda i: (0, 0)),
            pl.BlockSpec((1, D), lambda i: (0, 0)),
        ],
        out_specs=pl.BlockSpec((n, D), lambda i: (0, 0)),
        out_shape=jax.ShapeDtypeStruct((n, D), jnp.float32),
    )(aggf, aggf, degf, degf, xprev, root, bias2d)


def _sc_edge_call(y12, srcw, dstw, attrw, z32, z16, o16, nagg, nch, with_deg):
    rps = nagg // NS
    nblk = rps // CHUNK
    mesh = plsc.VectorSubcoreMesh(core_axis_name="c", subcore_axis_name="s")
    out_type = [jax.ShapeDtypeStruct((NC, nagg, D), jnp.float32)]
    scratch = [
        pltpu.VMEM((nch, CHUNK), jnp.int32),
        pltpu.VMEM((nch, CHUNK), jnp.int32),
        pltpu.VMEM((DEPTH, CHUNK), jnp.float32),
        pltpu.VMEM((DEPTH, CHUNK, 2 * D), jnp.float32),
        pltpu.VMEM((2, CHUNK, D), jnp.float32),
        pltpu.VMEM((CHUNK, 16), jnp.float32),
        pltpu.VMEM_SHARED((nagg, D), jnp.float32),
        pltpu.SemaphoreType.DMA((DEPTH,)),
        pltpu.SemaphoreType.DMA((DEPTH,)),
        pltpu.SemaphoreType.DMA((2,)),
        pltpu.SemaphoreType.DMA((2,)),
    ]
    if with_deg:
        out_type.append(jax.ShapeDtypeStruct((NC, nagg, 16), jnp.float32))
        scratch.append(pltpu.VMEM_SHARED((nagg, 16), jnp.float32))

    def body(y12_hbm, src_hbm, dst_hbm, attr_hbm, z32_hbm, z16_hbm, o16_hbm,
             *refs):
        if with_deg:
            (agg_out, deg_out, srcv, dstv, attrv, rowsv, msgv, onesv, aggS,
             sem_a, sem_g, sem_s, sem_d, degS) = refs
        else:
            (agg_out, srcv, dstv, attrv, rowsv, msgv, onesv, aggS,
             sem_a, sem_g, sem_s, sem_d) = refs
        c = lax.axis_index("c")
        s = lax.axis_index("s")
        w = c * NS + s

        pltpu.sync_copy(src_hbm.at[w], srcv)
        pltpu.sync_copy(dst_hbm.at[w], dstv)
        if with_deg:
            pltpu.sync_copy(o16_hbm, onesv)

        def start(j, b):
            pltpu.async_copy(attr_hbm.at[w * nch + j], attrv.at[b],
                             sem_a.at[b])
            pltpu.async_copy(y12_hbm.at[srcv.at[j]], rowsv.at[b],
                             sem_g.at[b])

        @pl.loop(0, nblk)
        def _(t):
            base = s * rps + t * CHUNK
            pltpu.sync_copy(z32_hbm, aggS.at[pl.ds(base, CHUNK)])
            if with_deg:
                pltpu.sync_copy(z16_hbm, degS.at[pl.ds(base, CHUNK)])

        plsc.subcore_barrier()

        for b in range(DEPTH):
            start(b, b)

        @pl.loop(0, nch)
        def _(j):
            b = lax.rem(j, DEPTH)
            mb = lax.rem(j, 2)
            pltpu.make_async_copy(attr_hbm.at[0], attrv.at[b],
                                  sem_a.at[b]).wait()
            pltpu.make_async_copy(y12_hbm.at[srcv.at[0]], rowsv.at[b],
                                  sem_g.at[b]).wait()
            rv = rowsv.at[b]
            av_ = attrv.at[b]
            mv = msgv.at[mb]

            @pl.when(j >= 2)
            def _():
                pltpu.make_async_copy(msgv.at[mb], aggS.at[dstv.at[0]],
                                      sem_s.at[mb]).wait()
                if with_deg:
                    pltpu.make_async_copy(onesv, degS.at[dstv.at[0]],
                                          sem_d.at[mb]).wait()

            @pl.loop(0, CHUNK, step=4)
            def _(k):
                for u in range(4):
                    ku = k + u
                    av = plsc.load_gather(av_, [jnp.full((16,), ku,
                                                         jnp.int32)])
                    mv[ku, 0:16] = av * rv[ku, 0:16] + rv[ku, 32:48]
                    mv[ku, 16:32] = av * rv[ku, 16:32] + rv[ku, 48:64]

            @pl.when(j + DEPTH < nch)
            def _():
                start(j + DEPTH, b)

            pltpu.async_copy(msgv.at[mb], aggS.at[dstv.at[j]], sem_s.at[mb],
                             add=True)
            if with_deg:
                pltpu.async_copy(onesv, degS.at[dstv.at[j]], sem_d.at[mb],
                                 add=True)

        for mb in range(2):
            pltpu.make_async_copy(msgv.at[mb], aggS.at[dstv.at[0]],
                                  sem_s.at[mb]).wait()
            if with_deg:
                pltpu.make_async_copy(onesv, degS.at[dstv.at[0]],
                                      sem_d.at[mb]).wait()

        plsc.subcore_barrier()

        @pl.loop(0, nblk)
        def _(t):
            base = s * rps + t * CHUNK
            pltpu.sync_copy(aggS.at[pl.ds(base, CHUNK)],
                            agg_out.at[c].at[pl.ds(base, CHUNK)])
            if with_deg:
                pltpu.sync_copy(degS.at[pl.ds(base, CHUNK)],
                                deg_out.at[c].at[pl.ds(base, CHUNK)])

    fn = pl.kernel(
        body, mesh=mesh, out_type=out_type, scratch_types=scratch,
        compiler_params=pltpu.CompilerParams(use_tc_tiling_on_sc=False,
                                             needs_layout_passes=False))
    return fn(y12, srcw, dstw, attrw, z32, z16, o16)


def kernel(x_u, x_v, edge_index, edge_attribute, i, dummy,
           Wu, bu, Wv, bv, Wl1, bl1, root, bias):
    n_u = x_u.shape[0]
    n_v = x_v.shape[0]
    n = n_u + n_v
    e = edge_index.shape[1]

    w1 = Wl1.reshape(D, D)
    b1 = bl1.reshape(D, D)
    w12 = jnp.concatenate([w1, b1], axis=1)
    w3 = jnp.concatenate([Wu, Wv, jnp.zeros((5, D), jnp.float32)], axis=0)
    bcat2 = jnp.stack([bu, bv], axis=0).reshape(2, 1, D)
    xcat = jnp.concatenate([
        jnp.pad(x_u, ((0, 0), (0, 7))),
        jnp.pad(x_v, ((0, 0), (1, 5))),
    ], axis=0)
    bias2d = bias.reshape(1, D)

    epad = -(-e // (NW * CHUNK)) * (NW * CHUNK)
    nch = epad // (NW * CHUNK)
    pad = epad - e
    nagg = -(-(n + 1) // (NS * CHUNK)) * (NS * CHUNK)
    src = jnp.concatenate([edge_index[0], jnp.zeros((pad,), jnp.int32)])
    dst = jnp.concatenate([edge_index[1], jnp.full((pad,), n, jnp.int32)])
    attr = jnp.concatenate([edge_attribute[:, 0],
                            jnp.zeros((pad,), jnp.float32)])
    srcw = src.reshape(NW, nch, CHUNK)
    dstw = dst.reshape(NW, nch, CHUNK)
    attrw = attr.reshape(NW * nch, CHUNK)
    z32 = jnp.zeros((CHUNK, D), jnp.float32)
    z16 = jnp.zeros((CHUNK, 16), jnp.float32)
    o16 = jnp.ones((CHUNK, 16), jnp.float32)

    xr0, y12_0 = _tca(xcat, w3, bcat2, w12, n_u)
    aggf0, degf = _sc_edge_call(y12_0, srcw, dstw, attrw, z32, z16, o16,
                                nagg, nch, True)
    xr1, y12_1 = _tcb(aggf0, degf, xr0, root, bias2d, w12)
    aggf1 = _sc_edge_call(y12_1, srcw, dstw, attrw, z32, z16, o16,
                          nagg, nch, False)[0]
    return _tcc(aggf1, degf, xr1, root, bias2d)


# speedup vs baseline: 13.0258x; 1.3127x over previous
import jax
import jax.numpy as jnp
from jax import lax
from jax.experimental import pallas as pl
from jax.experimental.pallas import tpu as pltpu
from jax.experimental.pallas import tpu_sc as plsc

D = 32
NC = 2
NS = 16
NW = NC * NS
CHUNK = 128
DEPTH = 6


def _make_tca_body(n_u):
    def _tca_body(xcat_ref, w3_ref, bcat_ref, xr_ref):
        n = xcat_ref.shape[0]
        row = jax.lax.broadcasted_iota(jnp.int32, (n, 1), 0)
        b = jnp.where(row < n_u, bcat_ref[0], bcat_ref[1])
        x0 = jnp.dot(xcat_ref[...], w3_ref[...],
                     preferred_element_type=jnp.float32) + b
        xr_ref[...] = jnp.maximum(x0, 0.0)
    return _tca_body


def _mean_agg(p0_ref, p1_ref, d0_ref, d1_ref, w1_ref, b1_ref):
    st = p0_ref[0] + p1_ref[0]
    deg = jnp.maximum(d0_ref[0, :, 0:1] + d1_ref[0, :, 0:1], 1.0)
    agg = (jnp.dot(st[:, D:], w1_ref[...], preferred_element_type=jnp.float32)
           + jnp.dot(st[:, :D], b1_ref[...],
                     preferred_element_type=jnp.float32))
    return agg / deg


def _tcb_body(p0_ref, p1_ref, d0_ref, d1_ref, xprev_ref, root_ref, bias_ref,
              w1_ref, b1_ref, xr_ref):
    agg = _mean_agg(p0_ref, p1_ref, d0_ref, d1_ref, w1_ref, b1_ref)
    x1 = agg + jnp.dot(xprev_ref[...], root_ref[...],
                       preferred_element_type=jnp.float32) + bias_ref[...]
    xr_ref[...] = jnp.maximum(x1, 0.0)


def _tcc_body(p0_ref, p1_ref, d0_ref, d1_ref, xprev_ref, root_ref, bias_ref,
              w1_ref, b1_ref, out_ref):
    agg = _mean_agg(p0_ref, p1_ref, d0_ref, d1_ref, w1_ref, b1_ref)
    out_ref[...] = agg + jnp.dot(xprev_ref[...], root_ref[...],
                                 preferred_element_type=jnp.float32) + bias_ref[...]


def _tca(xcat, w3, bcat2, n_u):
    n = xcat.shape[0]
    return pl.pallas_call(
        _make_tca_body(n_u),
        out_shape=jax.ShapeDtypeStruct((n, D), jnp.float32),
    )(xcat, w3, bcat2)


def _tc_layer(body, aggf, degf, xprev, root, bias2d, w1, b1):
    n = xprev.shape[0]
    specs = [
        pl.BlockSpec((1, n, 2 * D), lambda i: (0, 0, 0)),
        pl.BlockSpec((1, n, 2 * D), lambda i: (1, 0, 0)),
        pl.BlockSpec((1, n, 16), lambda i: (0, 0, 0)),
        pl.BlockSpec((1, n, 16), lambda i: (1, 0, 0)),
        pl.BlockSpec((n, D), lambda i: (0, 0)),
        pl.BlockSpec((D, D), lambda i: (0, 0)),
        pl.BlockSpec((1, D), lambda i: (0, 0)),
        pl.BlockSpec((D, D), lambda i: (0, 0)),
        pl.BlockSpec((D, D), lambda i: (0, 0)),
    ]
    return pl.pallas_call(
        body,
        grid=(1,),
        in_specs=specs,
        out_specs=pl.BlockSpec((n, D), lambda i: (0, 0)),
        out_shape=jax.ShapeDtypeStruct((n, D), jnp.float32),
    )(aggf, aggf, degf, degf, xprev, root, bias2d, w1, b1)


def _sc_edge_call(xr, srcw, dstw, attrw, z64, z16, o16, nagg, nch, with_deg):
    rps = nagg // NS
    nblk = rps // CHUNK
    mesh = plsc.VectorSubcoreMesh(core_axis_name="c", subcore_axis_name="s")
    out_type = [jax.ShapeDtypeStruct((NC, nagg, 2 * D), jnp.float32)]
    scratch = [
        pltpu.VMEM((nch, CHUNK), jnp.int32),
        pltpu.VMEM((nch, CHUNK), jnp.int32),
        pltpu.VMEM((DEPTH, CHUNK), jnp.float32),
        pltpu.VMEM((DEPTH, CHUNK, D), jnp.float32),
        pltpu.VMEM((2, CHUNK, 2 * D), jnp.float32),
        pltpu.VMEM((CHUNK, 16), jnp.float32),
        pltpu.VMEM_SHARED((nagg, 2 * D), jnp.float32),
        pltpu.SemaphoreType.DMA((DEPTH,)),
        pltpu.SemaphoreType.DMA((DEPTH,)),
        pltpu.SemaphoreType.DMA((2,)),
        pltpu.SemaphoreType.DMA((2,)),
    ]
    if with_deg:
        out_type.append(jax.ShapeDtypeStruct((NC, nagg, 16), jnp.float32))
        scratch.append(pltpu.VMEM_SHARED((nagg, 16), jnp.float32))

    def body(xr_hbm, src_hbm, dst_hbm, attr_hbm, z64_hbm, z16_hbm, o16_hbm,
             *refs):
        if with_deg:
            (agg_out, deg_out, srcv, dstv, attrv, rowsv, msgv, onesv, aggS,
             sem_a, sem_g, sem_s, sem_d, degS) = refs
        else:
            (agg_out, srcv, dstv, attrv, rowsv, msgv, onesv, aggS,
             sem_a, sem_g, sem_s, sem_d) = refs
        c = lax.axis_index("c")
        s = lax.axis_index("s")
        w = c * NS + s

        pltpu.sync_copy(src_hbm.at[w], srcv)
        pltpu.sync_copy(dst_hbm.at[w], dstv)
        if with_deg:
            pltpu.sync_copy(o16_hbm, onesv)

        def start(j, b):
            pltpu.async_copy(attr_hbm.at[w * nch + j], attrv.at[b],
                             sem_a.at[b])
            pltpu.async_copy(xr_hbm.at[srcv.at[j]], rowsv.at[b],
                             sem_g.at[b])

        @pl.loop(0, nblk)
        def _(t):
            base = s * rps + t * CHUNK
            pltpu.sync_copy(z64_hbm, aggS.at[pl.ds(base, CHUNK)])
            if with_deg:
                pltpu.sync_copy(z16_hbm, degS.at[pl.ds(base, CHUNK)])

        plsc.subcore_barrier()

        for b in range(DEPTH):
            start(b, b)

        @pl.loop(0, nch)
        def _(j):
            b = lax.rem(j, DEPTH)
            mb = lax.rem(j, 2)
            pltpu.make_async_copy(attr_hbm.at[0], attrv.at[b],
                                  sem_a.at[b]).wait()
            pltpu.make_async_copy(xr_hbm.at[srcv.at[0]], rowsv.at[b],
                                  sem_g.at[b]).wait()
            rv = rowsv.at[b]
            av_ = attrv.at[b]
            mv = msgv.at[mb]

            @pl.when(j >= 2)
            def _():
                pltpu.make_async_copy(msgv.at[mb], aggS.at[dstv.at[0]],
                                      sem_s.at[mb]).wait()
                if with_deg:
                    pltpu.make_async_copy(onesv, degS.at[dstv.at[0]],
                                          sem_d.at[mb]).wait()

            @pl.loop(0, CHUNK, step=4)
            def _(k):
                for u in range(4):
                    ku = k + u
                    av = plsc.load_gather(av_, [jnp.full((16,), ku,
                                                         jnp.int32)])
                    xa = rv[ku, 0:16]
                    xb = rv[ku, 16:32]
                    mv[ku, 0:16] = xa
                    mv[ku, 16:32] = xb
                    mv[ku, 32:48] = av * xa
                    mv[ku, 48:64] = av * xb

            @pl.when(j + DEPTH < nch)
            def _():
                start(j + DEPTH, b)

            pltpu.async_copy(msgv.at[mb], aggS.at[dstv.at[j]], sem_s.at[mb],
                             add=True)
            if with_deg:
                pltpu.async_copy(onesv, degS.at[dstv.at[j]], sem_d.at[mb],
                                 add=True)

        for mb in range(2):
            pltpu.make_async_copy(msgv.at[mb], aggS.at[dstv.at[0]],
                                  sem_s.at[mb]).wait()
            if with_deg:
                pltpu.make_async_copy(onesv, degS.at[dstv.at[0]],
                                      sem_d.at[mb]).wait()

        plsc.subcore_barrier()

        @pl.loop(0, nblk)
        def _(t):
            base = s * rps + t * CHUNK
            pltpu.sync_copy(aggS.at[pl.ds(base, CHUNK)],
                            agg_out.at[c].at[pl.ds(base, CHUNK)])
            if with_deg:
                pltpu.sync_copy(degS.at[pl.ds(base, CHUNK)],
                                deg_out.at[c].at[pl.ds(base, CHUNK)])

    fn = pl.kernel(
        body, mesh=mesh, out_type=out_type, scratch_types=scratch,
        compiler_params=pltpu.CompilerParams(use_tc_tiling_on_sc=False,
                                             needs_layout_passes=False))
    return fn(xr, srcw, dstw, attrw, z64, z16, o16)


def kernel(x_u, x_v, edge_index, edge_attribute, i, dummy,
           Wu, bu, Wv, bv, Wl1, bl1, root, bias):
    n_u = x_u.shape[0]
    n_v = x_v.shape[0]
    n = n_u + n_v
    e = edge_index.shape[1]

    w1 = Wl1.reshape(D, D)
    b1 = bl1.reshape(D, D)
    w3 = jnp.concatenate([Wu, Wv, jnp.zeros((5, D), jnp.float32)], axis=0)
    bcat2 = jnp.stack([bu, bv], axis=0).reshape(2, 1, D)
    xcat = jnp.concatenate([
        jnp.pad(x_u, ((0, 0), (0, 7))),
        jnp.pad(x_v, ((0, 0), (1, 5))),
    ], axis=0)
    bias2d = bias.reshape(1, D)

    epad = -(-e // (NW * CHUNK)) * (NW * CHUNK)
    nch = epad // (NW * CHUNK)
    pad = epad - e
    nagg = -(-(n + 1) // (NS * CHUNK)) * (NS * CHUNK)
    src = jnp.concatenate([edge_index[0], jnp.zeros((pad,), jnp.int32)])
    dst = jnp.concatenate([edge_index[1], jnp.full((pad,), n, jnp.int32)])
    attr = jnp.concatenate([edge_attribute[:, 0],
                            jnp.zeros((pad,), jnp.float32)])
    srcw = src.reshape(NW, nch, CHUNK)
    dstw = dst.reshape(NW, nch, CHUNK)
    attrw = attr.reshape(NW * nch, CHUNK)
    z64 = jnp.zeros((CHUNK, 2 * D), jnp.float32)
    z16 = jnp.zeros((CHUNK, 16), jnp.float32)
    o16 = jnp.ones((CHUNK, 16), jnp.float32)

    xr0 = _tca(xcat, w3, bcat2, n_u)
    aggf0, degf = _sc_edge_call(xr0, srcw, dstw, attrw, z64, z16, o16,
                                nagg, nch, True)
    xr1 = _tc_layer(_tcb_body, aggf0, degf, xr0, root, bias2d, w1, b1)
    aggf1 = _sc_edge_call(xr1, srcw, dstw, attrw, z64, z16, o16,
                          nagg, nch, False)[0]
    return _tc_layer(_tcc_body, aggf1, degf, xr1, root, bias2d, w1, b1)


# speedup vs baseline: 15.5186x; 1.1914x over previous
import jax
import jax.numpy as jnp
from jax import lax
from jax.experimental import pallas as pl
from jax.experimental.pallas import tpu as pltpu
from jax.experimental.pallas import tpu_sc as plsc

D = 32
NC = 2
NS = 16
NW = NC * NS
CHUNK = 128
DEPTH = 6


def _make_tca_body(n_u):
    def _tca_body(xcat_ref, w3_ref, bcat_ref, xr_ref):
        n = xcat_ref.shape[0]
        row = jax.lax.broadcasted_iota(jnp.int32, (n, 1), 0)
        b = jnp.where(row < n_u, bcat_ref[0], bcat_ref[1])
        x0 = jnp.dot(xcat_ref[...], w3_ref[...],
                     preferred_element_type=jnp.float32) + b
        xr_ref[...] = jnp.maximum(x0, 0.0)
    return _tca_body


def _mean_agg(p0_ref, p1_ref, d0_ref, d1_ref, w1_ref, b1_ref):
    st = p0_ref[0] + p1_ref[0]
    deg = jnp.maximum(d0_ref[0, :, 0:1] + d1_ref[0, :, 0:1], 1.0)
    agg = (jnp.dot(st[:, D:], w1_ref[...], preferred_element_type=jnp.float32)
           + jnp.dot(st[:, :D], b1_ref[...],
                     preferred_element_type=jnp.float32))
    return agg / deg


def _tcb_body(p0_ref, p1_ref, d0_ref, d1_ref, xprev_ref, root_ref, bias_ref,
              w1_ref, b1_ref, xr_ref):
    agg = _mean_agg(p0_ref, p1_ref, d0_ref, d1_ref, w1_ref, b1_ref)
    x1 = agg + jnp.dot(xprev_ref[...], root_ref[...],
                       preferred_element_type=jnp.float32) + bias_ref[...]
    xr_ref[...] = jnp.maximum(x1, 0.0)


def _tcc_body(p0_ref, p1_ref, d0_ref, d1_ref, xprev_ref, root_ref, bias_ref,
              w1_ref, b1_ref, out_ref):
    agg = _mean_agg(p0_ref, p1_ref, d0_ref, d1_ref, w1_ref, b1_ref)
    out_ref[...] = agg + jnp.dot(xprev_ref[...], root_ref[...],
                                 preferred_element_type=jnp.float32) + bias_ref[...]


def _tca(xcat, w3, bcat2, n_u):
    n = xcat.shape[0]
    return pl.pallas_call(
        _make_tca_body(n_u),
        out_shape=jax.ShapeDtypeStruct((n, D), jnp.float32),
    )(xcat, w3, bcat2)


def _tc_layer(body, aggf, degf, xprev, root, bias2d, w1, b1):
    n = xprev.shape[0]
    specs = [
        pl.BlockSpec((1, n, 2 * D), lambda i: (0, 0, 0)),
        pl.BlockSpec((1, n, 2 * D), lambda i: (1, 0, 0)),
        pl.BlockSpec((1, n, 16), lambda i: (0, 0, 0)),
        pl.BlockSpec((1, n, 16), lambda i: (1, 0, 0)),
        pl.BlockSpec((n, D), lambda i: (0, 0)),
        pl.BlockSpec((D, D), lambda i: (0, 0)),
        pl.BlockSpec((1, D), lambda i: (0, 0)),
        pl.BlockSpec((D, D), lambda i: (0, 0)),
        pl.BlockSpec((D, D), lambda i: (0, 0)),
    ]
    return pl.pallas_call(
        body,
        grid=(1,),
        in_specs=specs,
        out_specs=pl.BlockSpec((n, D), lambda i: (0, 0)),
        out_shape=jax.ShapeDtypeStruct((n, D), jnp.float32),
    )(aggf, aggf, degf, degf, xprev, root, bias2d, w1, b1)


def _sc_edge_call(xr, srcw, dstw, attrw, z64, z16, o16, nagg, nch, with_deg):
    rps = nagg // NS
    nblk = rps // CHUNK
    mesh = plsc.VectorSubcoreMesh(core_axis_name="c", subcore_axis_name="s")
    out_type = [jax.ShapeDtypeStruct((NC, nagg, 2 * D), jnp.float32)]
    scratch = [
        pltpu.VMEM((nch, CHUNK), jnp.int32),
        pltpu.VMEM((nch, CHUNK), jnp.int32),
        pltpu.VMEM((DEPTH, CHUNK), jnp.float32),
        pltpu.VMEM((DEPTH, CHUNK, D), jnp.float32),
        pltpu.VMEM((2, CHUNK, 2 * D), jnp.float32),
        pltpu.VMEM((CHUNK, 16), jnp.float32),
        pltpu.VMEM_SHARED((nagg, 2 * D), jnp.float32),
        pltpu.SemaphoreType.DMA((DEPTH,)),
        pltpu.SemaphoreType.DMA((DEPTH,)),
        pltpu.SemaphoreType.DMA((2,)),
        pltpu.SemaphoreType.DMA((2,)),
    ]
    if with_deg:
        out_type.append(jax.ShapeDtypeStruct((NC, nagg, 16), jnp.float32))
        scratch.append(pltpu.VMEM_SHARED((nagg, 16), jnp.float32))

    def body(xr_hbm, src_hbm, dst_hbm, attr_hbm, z64_hbm, z16_hbm, o16_hbm,
             *refs):
        if with_deg:
            (agg_out, deg_out, srcv, dstv, attrv, rowsv, msgv, onesv, aggS,
             sem_a, sem_g, sem_s, sem_d, degS) = refs
        else:
            (agg_out, srcv, dstv, attrv, rowsv, msgv, onesv, aggS,
             sem_a, sem_g, sem_s, sem_d) = refs
        c = lax.axis_index("c")
        s = lax.axis_index("s")
        w = c * NS + s

        pltpu.sync_copy(src_hbm.at[w], srcv)
        pltpu.sync_copy(dst_hbm.at[w], dstv)
        if with_deg:
            pltpu.sync_copy(o16_hbm, onesv)

        def start(j, b):
            pltpu.async_copy(attr_hbm.at[w * nch + j], attrv.at[b],
                             sem_a.at[b])
            pltpu.async_copy(xr_hbm.at[srcv.at[j]], rowsv.at[b],
                             sem_g.at[b])

        @pl.loop(0, nblk)
        def _(t):
            base = s * rps + t * CHUNK
            pltpu.sync_copy(z64_hbm, aggS.at[pl.ds(base, CHUNK)])
            if with_deg:
                pltpu.sync_copy(z16_hbm, degS.at[pl.ds(base, CHUNK)])

        plsc.subcore_barrier()

        for b in range(DEPTH):
            start(b, b)

        @pl.loop(0, nch)
        def _(j):
            b = lax.rem(j, DEPTH)
            mb = lax.rem(j, 2)
            pltpu.make_async_copy(attr_hbm.at[0], attrv.at[b],
                                  sem_a.at[b]).wait()
            pltpu.make_async_copy(xr_hbm.at[srcv.at[0]], rowsv.at[b],
                                  sem_g.at[b]).wait()
            rv = rowsv.at[b]
            av_ = attrv.at[b]
            mv = msgv.at[mb]

            @pl.when(j >= 2)
            def _():
                pltpu.make_async_copy(msgv.at[mb], aggS.at[dstv.at[0]],
                                      sem_s.at[mb]).wait()
                if with_deg:
                    pltpu.make_async_copy(onesv, degS.at[dstv.at[0]],
                                          sem_d.at[mb]).wait()

            @pl.loop(0, CHUNK, step=4)
            def _(k):
                for u in range(4):
                    ku = k + u
                    av = plsc.load_gather(av_, [jnp.full((16,), ku,
                                                         jnp.int32)])
                    xa = rv[ku, 0:16]
                    xb = rv[ku, 16:32]
                    mv[ku, 0:16] = xa
                    mv[ku, 16:32] = xb
                    mv[ku, 32:48] = av * xa
                    mv[ku, 48:64] = av * xb

            @pl.when(j + DEPTH < nch)
            def _():
                start(j + DEPTH, b)

            pltpu.async_copy(msgv.at[mb], aggS.at[dstv.at[j]], sem_s.at[mb],
                             add=True)
            if with_deg:
                pltpu.async_copy(onesv, degS.at[dstv.at[j]], sem_d.at[mb],
                                 add=True)

        for mb in range(2):
            pltpu.make_async_copy(msgv.at[mb], aggS.at[dstv.at[0]],
                                  sem_s.at[mb]).wait()
            if with_deg:
                pltpu.make_async_copy(onesv, degS.at[dstv.at[0]],
                                      sem_d.at[mb]).wait()

        plsc.subcore_barrier()

        @pl.loop(0, nblk)
        def _(t):
            base = s * rps + t * CHUNK
            pltpu.sync_copy(aggS.at[pl.ds(base, CHUNK)],
                            agg_out.at[c].at[pl.ds(base, CHUNK)])
            if with_deg:
                pltpu.sync_copy(degS.at[pl.ds(base, CHUNK)],
                                deg_out.at[c].at[pl.ds(base, CHUNK)])

    fn = pl.kernel(
        body, mesh=mesh, out_type=out_type, scratch_types=scratch,
        compiler_params=pltpu.CompilerParams(use_tc_tiling_on_sc=False,
                                             needs_layout_passes=False))
    return fn(xr, srcw, dstw, attrw, z64, z16, o16)


def kernel(x_u, x_v, edge_index, edge_attribute, i, dummy,
           Wu, bu, Wv, bv, Wl1, bl1, root, bias):
    n_u = x_u.shape[0]
    n_v = x_v.shape[0]
    n = n_u + n_v
    e = edge_index.shape[1]

    w1 = Wl1.reshape(D, D)
    b1 = bl1.reshape(D, D)
    w3 = jnp.concatenate([Wu, Wv, jnp.zeros((5, D), jnp.float32)], axis=0)
    bcat2 = jnp.stack([bu, bv], axis=0).reshape(2, 1, D)
    xcat = jnp.concatenate([
        jnp.pad(x_u, ((0, 0), (0, 7))),
        jnp.pad(x_v, ((0, 0), (1, 5))),
    ], axis=0)
    bias2d = bias.reshape(1, D)

    epad = -(-e // (NW * CHUNK)) * (NW * CHUNK)
    nch = epad // (NW * CHUNK)
    pad = epad - e
    nagg = -(-(n + 1) // (NS * CHUNK)) * (NS * CHUNK)
    pad_src = jnp.arange(pad, dtype=jnp.int32) % n
    pad_dst = n + jnp.arange(pad, dtype=jnp.int32) % (nagg - n)
    src = jnp.concatenate([edge_index[0], pad_src])
    dst = jnp.concatenate([edge_index[1], pad_dst])
    attr = jnp.concatenate([edge_attribute[:, 0],
                            jnp.zeros((pad,), jnp.float32)])
    srcw = src.reshape(NW, nch, CHUNK)
    dstw = dst.reshape(NW, nch, CHUNK)
    attrw = attr.reshape(NW * nch, CHUNK)
    z64 = jnp.zeros((CHUNK, 2 * D), jnp.float32)
    z16 = jnp.zeros((CHUNK, 16), jnp.float32)
    o16 = jnp.ones((CHUNK, 16), jnp.float32)

    xr0 = _tca(xcat, w3, bcat2, n_u)
    aggf0, degf = _sc_edge_call(xr0, srcw, dstw, attrw, z64, z16, o16,
                                nagg, nch, True)
    xr1 = _tc_layer(_tcb_body, aggf0, degf, xr0, root, bias2d, w1, b1)
    aggf1 = _sc_edge_call(xr1, srcw, dstw, attrw, z64, z16, o16,
                          nagg, nch, False)[0]
    return _tc_layer(_tcc_body, aggf1, degf, xr1, root, bias2d, w1, b1)


# speedup vs baseline: 15.6031x; 1.0054x over previous
import jax
import jax.numpy as jnp
from jax import lax
from jax.experimental import pallas as pl
from jax.experimental.pallas import tpu as pltpu
from jax.experimental.pallas import tpu_sc as plsc

D = 32
NC = 2
NS = 16
NW = NC * NS
CHUNK = 128
DEPTH = 6


def _make_tca_body(n_u, blk):
    def _tca_body(xcat_ref, w3_ref, bcat_ref, xr_ref):
        row = (pl.program_id(0) * blk
               + jax.lax.broadcasted_iota(jnp.int32, (blk, 1), 0))
        b = jnp.where(row < n_u, bcat_ref[0], bcat_ref[1])
        x0 = jnp.dot(xcat_ref[...], w3_ref[...],
                     preferred_element_type=jnp.float32) + b
        xr_ref[...] = jnp.maximum(x0, 0.0)
    return _tca_body


def _mean_agg(p0_ref, p1_ref, d0_ref, d1_ref, w1_ref, b1_ref):
    st = p0_ref[0] + p1_ref[0]
    deg = jnp.maximum(d0_ref[0, :, 0:1] + d1_ref[0, :, 0:1], 1.0)
    agg = (jnp.dot(st[:, D:], w1_ref[...], preferred_element_type=jnp.float32)
           + jnp.dot(st[:, :D], b1_ref[...],
                     preferred_element_type=jnp.float32))
    return agg / deg


def _tcb_body(p0_ref, p1_ref, d0_ref, d1_ref, xprev_ref, root_ref, bias_ref,
              w1_ref, b1_ref, xr_ref):
    agg = _mean_agg(p0_ref, p1_ref, d0_ref, d1_ref, w1_ref, b1_ref)
    x1 = agg + jnp.dot(xprev_ref[...], root_ref[...],
                       preferred_element_type=jnp.float32) + bias_ref[...]
    xr_ref[...] = jnp.maximum(x1, 0.0)


def _tcc_body(p0_ref, p1_ref, d0_ref, d1_ref, xprev_ref, root_ref, bias_ref,
              w1_ref, b1_ref, out_ref):
    agg = _mean_agg(p0_ref, p1_ref, d0_ref, d1_ref, w1_ref, b1_ref)
    out_ref[...] = agg + jnp.dot(xprev_ref[...], root_ref[...],
                                 preferred_element_type=jnp.float32) + bias_ref[...]


def _tca(xcat, w3, bcat2, n_u):
    n = xcat.shape[0]
    blk = n // 5
    return pl.pallas_call(
        _make_tca_body(n_u, blk),
        grid=(5,),
        in_specs=[
            pl.BlockSpec((blk, 8), lambda i: (i, 0)),
            pl.BlockSpec((8, D), lambda i: (0, 0)),
            pl.BlockSpec((2, 1, D), lambda i: (0, 0, 0)),
        ],
        out_specs=pl.BlockSpec((blk, D), lambda i: (i, 0)),
        out_shape=jax.ShapeDtypeStruct((n, D), jnp.float32),
    )(xcat, w3, bcat2)


def _tc_layer(body, aggf, degf, xprev, root, bias2d, w1, b1):
    n = xprev.shape[0]
    blk = n // 5
    specs = [
        pl.BlockSpec((1, blk, 2 * D), lambda i: (0, i, 0)),
        pl.BlockSpec((1, blk, 2 * D), lambda i: (1, i, 0)),
        pl.BlockSpec((1, blk, 16), lambda i: (0, i, 0)),
        pl.BlockSpec((1, blk, 16), lambda i: (1, i, 0)),
        pl.BlockSpec((blk, D), lambda i: (i, 0)),
        pl.BlockSpec((D, D), lambda i: (0, 0)),
        pl.BlockSpec((1, D), lambda i: (0, 0)),
        pl.BlockSpec((D, D), lambda i: (0, 0)),
        pl.BlockSpec((D, D), lambda i: (0, 0)),
    ]
    return pl.pallas_call(
        body,
        grid=(5,),
        in_specs=specs,
        out_specs=pl.BlockSpec((blk, D), lambda i: (i, 0)),
        out_shape=jax.ShapeDtypeStruct((n, D), jnp.float32),
    )(aggf, aggf, degf, degf, xprev, root, bias2d, w1, b1)


def _sc_edge_call(xr, srcw, dstw, attrw, z64, z16, o16, nagg, nch, with_deg):
    rps = nagg // NS
    nblk = rps // CHUNK
    mesh = plsc.VectorSubcoreMesh(core_axis_name="c", subcore_axis_name="s")
    out_type = [jax.ShapeDtypeStruct((NC, nagg, 2 * D), jnp.float32)]
    scratch = [
        pltpu.VMEM((nch, CHUNK), jnp.int32),
        pltpu.VMEM((nch, CHUNK), jnp.int32),
        pltpu.VMEM((DEPTH, CHUNK), jnp.float32),
        pltpu.VMEM((DEPTH, CHUNK, D), jnp.float32),
        pltpu.VMEM((2, CHUNK, 2 * D), jnp.float32),
        pltpu.VMEM((CHUNK, 16), jnp.float32),
        pltpu.VMEM_SHARED((nagg, 2 * D), jnp.float32),
        pltpu.SemaphoreType.DMA((DEPTH,)),
        pltpu.SemaphoreType.DMA((DEPTH,)),
        pltpu.SemaphoreType.DMA((2,)),
        pltpu.SemaphoreType.DMA((2,)),
    ]
    if with_deg:
        out_type.append(jax.ShapeDtypeStruct((NC, nagg, 16), jnp.float32))
        scratch.append(pltpu.VMEM_SHARED((nagg, 16), jnp.float32))

    def body(xr_hbm, src_hbm, dst_hbm, attr_hbm, z64_hbm, z16_hbm, o16_hbm,
             *refs):
        if with_deg:
            (agg_out, deg_out, srcv, dstv, attrv, rowsv, msgv, onesv, aggS,
             sem_a, sem_g, sem_s, sem_d, degS) = refs
        else:
            (agg_out, srcv, dstv, attrv, rowsv, msgv, onesv, aggS,
             sem_a, sem_g, sem_s, sem_d) = refs
        c = lax.axis_index("c")
        s = lax.axis_index("s")
        w = c * NS + s

        pltpu.sync_copy(src_hbm.at[w], srcv)
        pltpu.sync_copy(dst_hbm.at[w], dstv)
        if with_deg:
            pltpu.sync_copy(o16_hbm, onesv)

        def start(j, b):
            pltpu.async_copy(attr_hbm.at[w * nch + j], attrv.at[b],
                             sem_a.at[b])
            pltpu.async_copy(xr_hbm.at[srcv.at[j]], rowsv.at[b],
                             sem_g.at[b])

        @pl.loop(0, nblk)
        def _(t):
            base = s * rps + t * CHUNK
            pltpu.sync_copy(z64_hbm, aggS.at[pl.ds(base, CHUNK)])
            if with_deg:
                pltpu.sync_copy(z16_hbm, degS.at[pl.ds(base, CHUNK)])

        plsc.subcore_barrier()

        for b in range(DEPTH):
            start(b, b)

        @pl.loop(0, nch)
        def _(j):
            b = lax.rem(j, DEPTH)
            mb = lax.rem(j, 2)
            pltpu.make_async_copy(attr_hbm.at[0], attrv.at[b],
                                  sem_a.at[b]).wait()
            pltpu.make_async_copy(xr_hbm.at[srcv.at[0]], rowsv.at[b],
                                  sem_g.at[b]).wait()
            rv = rowsv.at[b]
            av_ = attrv.at[b]
            mv = msgv.at[mb]

            @pl.when(j >= 2)
            def _():
                pltpu.make_async_copy(msgv.at[mb], aggS.at[dstv.at[0]],
                                      sem_s.at[mb]).wait()
                if with_deg:
                    pltpu.make_async_copy(onesv, degS.at[dstv.at[0]],
                                          sem_d.at[mb]).wait()

            @pl.loop(0, CHUNK, step=4)
            def _(k):
                for u in range(4):
                    ku = k + u
                    av = plsc.load_gather(av_, [jnp.full((16,), ku,
                                                         jnp.int32)])
                    xa = rv[ku, 0:16]
                    xb = rv[ku, 16:32]
                    mv[ku, 0:16] = xa
                    mv[ku, 16:32] = xb
                    mv[ku, 32:48] = av * xa
                    mv[ku, 48:64] = av * xb

            @pl.when(j + DEPTH < nch)
            def _():
                start(j + DEPTH, b)

            pltpu.async_copy(msgv.at[mb], aggS.at[dstv.at[j]], sem_s.at[mb],
                             add=True)
            if with_deg:
                pltpu.async_copy(onesv, degS.at[dstv.at[j]], sem_d.at[mb],
                                 add=True)

        for mb in range(2):
            pltpu.make_async_copy(msgv.at[mb], aggS.at[dstv.at[0]],
                                  sem_s.at[mb]).wait()
            if with_deg:
                pltpu.make_async_copy(onesv, degS.at[dstv.at[0]],
                                      sem_d.at[mb]).wait()

        plsc.subcore_barrier()

        @pl.loop(0, nblk)
        def _(t):
            base = s * rps + t * CHUNK
            pltpu.sync_copy(aggS.at[pl.ds(base, CHUNK)],
                            agg_out.at[c].at[pl.ds(base, CHUNK)])
            if with_deg:
                pltpu.sync_copy(degS.at[pl.ds(base, CHUNK)],
                                deg_out.at[c].at[pl.ds(base, CHUNK)])

    fn = pl.kernel(
        body, mesh=mesh, out_type=out_type, scratch_types=scratch,
        compiler_params=pltpu.CompilerParams(use_tc_tiling_on_sc=False,
                                             needs_layout_passes=False,
                                             disable_bounds_checks=True))
    return fn(xr, srcw, dstw, attrw, z64, z16, o16)


def kernel(x_u, x_v, edge_index, edge_attribute, i, dummy,
           Wu, bu, Wv, bv, Wl1, bl1, root, bias):
    n_u = x_u.shape[0]
    n_v = x_v.shape[0]
    n = n_u + n_v
    e = edge_index.shape[1]

    w1 = Wl1.reshape(D, D)
    b1 = bl1.reshape(D, D)
    w3 = jnp.concatenate([Wu, Wv, jnp.zeros((5, D), jnp.float32)], axis=0)
    bcat2 = jnp.stack([bu, bv], axis=0).reshape(2, 1, D)
    xcat = jnp.concatenate([
        jnp.pad(x_u, ((0, 0), (0, 7))),
        jnp.pad(x_v, ((0, 0), (1, 5))),
    ], axis=0)
    bias2d = bias.reshape(1, D)

    epad = -(-e // (NW * CHUNK)) * (NW * CHUNK)
    nch = epad // (NW * CHUNK)
    pad = epad - e
    nagg = -(-(n + 1) // (NS * CHUNK)) * (NS * CHUNK)
    pad_src = jnp.arange(pad, dtype=jnp.int32) % n
    pad_dst = n + jnp.arange(pad, dtype=jnp.int32) % (nagg - n)
    src = jnp.concatenate([edge_index[0], pad_src])
    dst = jnp.concatenate([edge_index[1], pad_dst])
    attr = jnp.concatenate([edge_attribute[:, 0],
                            jnp.zeros((pad,), jnp.float32)])
    srcw = src.reshape(NW, nch, CHUNK)
    dstw = dst.reshape(NW, nch, CHUNK)
    attrw = attr.reshape(NW * nch, CHUNK)
    z64 = jnp.zeros((CHUNK, 2 * D), jnp.float32)
    z16 = jnp.zeros((CHUNK, 16), jnp.float32)
    o16 = jnp.ones((CHUNK, 16), jnp.float32)

    xr0 = _tca(xcat, w3, bcat2, n_u)
    aggf0, degf = _sc_edge_call(xr0, srcw, dstw, attrw, z64, z16, o16,
                                nagg, nch, True)
    xr1 = _tc_layer(_tcb_body, aggf0, degf, xr0, root, bias2d, w1, b1)
    aggf1 = _sc_edge_call(xr1, srcw, dstw, attrw, z64, z16, o16,
                          nagg, nch, False)[0]
    return _tc_layer(_tcc_body, aggf1, degf, xr1, root, bias2d, w1, b1)
